# Initial kernel scaffold; baseline (speedup 1.0000x reference)
import jax
import jax.numpy as jnp
from jax.experimental import pallas as pl


def kernel(edge_index, node_embeddings, W, b, curvature):
    raise NotImplementedError("write your pallas kernel here")


# speedup vs baseline: 5.8435x; 5.8435x over previous
import functools

import jax
import jax.numpy as jnp
from jax import lax
from jax.experimental import pallas as pl
from jax.experimental.pallas import tpu as pltpu
from jax.experimental.pallas import tpu_sc as plsc

_N = 10000
_D = 128
_NC = 2
_NS = 16
_NW = _NC * _NS
_K = 128
_ACC_R = 10240
_ZR = _ACC_R // _NS
_HR = _ACC_R // _K


def _transform_body(x_ref, w_ref, b_ref, c_ref, o_ref):
    x = x_ref[...]
    c = c_ref[0, 0]
    sc = jnp.sqrt(c)
    nrm = jnp.sqrt(jnp.sum(x * x, axis=1, keepdims=True))
    z = sc * nrm
    atanh_z = 0.5 * jnp.log((1.0 + z) / (1.0 - z))
    t = (2.0 / sc) * atanh_z * x / nrm
    o_ref[...] = lax.dot_general(t, w_ref[...], (((1,), (1,)), ((), ())),
                                 preferred_element_type=jnp.float32) + b_ref[...]


def _combine_body(p_ref, h_ref, c_ref, o_ref):
    s = p_ref[0] + p_ref[1]
    cnt = jnp.sum(h_ref[...], axis=1, keepdims=True)
    v = s / jnp.maximum(cnt, 1.0)
    c = c_ref[0, 0]
    sc = jnp.sqrt(c)
    nrm = jnp.sqrt(jnp.sum(v * v, axis=1, keepdims=True))
    o_ref[...] = jnp.tanh(sc * nrm / 2.0) * v / (sc * nrm)


def _aggregate_body(src_hbm, dst_hbm, table_hbm, zeros_hbm,
                    out_hbm, cnt_hbm, sidx, didx, rows, hist, acc, sem):
    cid = lax.axis_index("c")
    sid = lax.axis_index("s")
    w = sid * _NC + cid

    pltpu.sync_copy(zeros_hbm, acc.at[pl.ds(sid * _ZR, _ZR)])
    pltpu.sync_copy(zeros_hbm.at[pl.ds(0, _HR)], hist)
    plsc.subcore_barrier()

    pltpu.sync_copy(src_hbm.at[w], sidx)
    pltpu.sync_copy(dst_hbm.at[w], didx)

    nchunks = sidx.shape[0]

    def body(i, carry):
        cp = pltpu.async_copy(table_hbm.at[sidx.at[i]], rows, sem)

        for j in range(_K // 16):
            d16 = didx[i, pl.ds(j * 16, 16)]
            cnt, last = plsc.scan_count(d16)
            plsc.addupdate_scatter(
                hist,
                [lax.shift_right_logical(d16, 7), lax.bitwise_and(d16, 127)],
                cnt.astype(jnp.float32),
                mask=last)
        cp.wait()
        pltpu.sync_copy(rows, acc.at[didx.at[i]], add=True)
        return carry

    lax.fori_loop(0, nchunks, body, 0)
    plsc.subcore_barrier()

    pltpu.sync_copy(acc.at[pl.ds(sid * _ZR, _ZR)],
                    out_hbm.at[pl.ds(cid * _ACC_R + sid * _ZR, _ZR)])
    pltpu.sync_copy(hist, cnt_hbm.at[w])


def kernel(edge_index, node_embeddings, W, b, curvature):
    n, d = node_embeddings.shape
    e = edge_index.shape[1]
    src = edge_index[0].astype(jnp.int32)
    dst = edge_index[1].astype(jnp.int32)

    ew = -(-e // (_NW * _K)) * _K
    e_pad = ew * _NW
    if e_pad != e:
        src = jnp.concatenate([src, jnp.zeros((e_pad - e,), jnp.int32)])
        dst = jnp.concatenate([dst, jnp.full((e_pad - e,), n, jnp.int32)])
    src3 = src.reshape(_NW, ew // _K, _K)
    dst3 = dst.reshape(_NW, ew // _K, _K)

    c2 = curvature.reshape(1, 1)
    b2 = b.reshape(1, d)

    bn = 2000
    table = pl.pallas_call(
        _transform_body,
        grid=(n // bn,),
        in_specs=[
            pl.BlockSpec((bn, d), lambda i: (i, 0)),
            pl.BlockSpec((d, d), lambda i: (0, 0)),
            pl.BlockSpec((1, d), lambda i: (0, 0)),
            pl.BlockSpec((1, 1), lambda i: (0, 0)),
        ],
        out_specs=pl.BlockSpec((bn, d), lambda i: (i, 0)),
        out_shape=jax.ShapeDtypeStruct((n, d), jnp.float32),
    )(node_embeddings, W, b2, c2)

    zeros = jnp.zeros((_ZR, _D), jnp.float32)

    mesh = plsc.VectorSubcoreMesh(core_axis_name="c", subcore_axis_name="s")
    agg = functools.partial(
        pl.kernel,
        out_type=(
            jax.ShapeDtypeStruct((_NC * _ACC_R, _D), jnp.float32),
            jax.ShapeDtypeStruct((_NW, _HR, _K), jnp.float32),
        ),
        mesh=mesh,
        compiler_params=pltpu.CompilerParams(needs_layout_passes=False),
        scratch_types=[
            pltpu.VMEM((ew // _K, _K), jnp.int32),
            pltpu.VMEM((ew // _K, _K), jnp.int32),
            pltpu.VMEM((_K, _D), jnp.float32),
            pltpu.VMEM((_HR, _K), jnp.float32),
            pltpu.VMEM_SHARED((_ACC_R, _D), jnp.float32),
            pltpu.SemaphoreType.DMA,
        ],
    )(_aggregate_body)
    partial_sums, hists = agg(src3, dst3, table, zeros)

    bc = 5120
    out = pl.pallas_call(
        _combine_body,
        grid=(_ACC_R // bc,),
        in_specs=[
            pl.BlockSpec((_NC, bc, _D), lambda i: (0, i, 0)),
            pl.BlockSpec((bc, _NW), lambda i: (i, 0)),
            pl.BlockSpec((1, 1), lambda i: (0, 0)),
        ],
        out_specs=pl.BlockSpec((bc, d), lambda i: (i, 0)),
        out_shape=jax.ShapeDtypeStruct((n, d), jnp.float32),
    )(partial_sums.reshape(_NC, _ACC_R, _D),
      hists.reshape(_NW, _ACC_R).T, c2)

    return out
